# baseline (speedup 1.0000x reference)
import functools

import jax
import jax.numpy as jnp
from jax import lax
from jax.experimental import pallas as pl
from jax.experimental.pallas import tpu as pltpu
from jax.experimental.pallas import tpu_sc as plsc

B = 4
S = 2048
D = 768
NW = 32
S_PER_W = S // NW
CG = 32
H = S_PER_W // CG
NCHUNK = B * H
NBUF = 3
NAHEAD = 2
VECS = D // 16


def _body(x_hbm, table_hbm, pe_hbm, out_hbm, idx_v, pe_v, rows_v,
          sem_i, sem_p, sems_g, sems_o):
    cid = lax.axis_index("c")
    sid = lax.axis_index("s")
    wid = cid * 16 + sid
    s0 = wid * S_PER_W

    cps_i = [
        pltpu.async_copy(x_hbm.at[b, pl.ds(s0, S_PER_W)], idx_v.at[b], sem_i)
        for b in range(B)
    ]
    cp_p = pltpu.async_copy(pe_hbm.at[pl.ds(s0, S_PER_W)], pe_v, sem_p)
    for cp in cps_i:
        cp.wait()

    def _bh(c):
        if isinstance(c, int):
            return c // H, c % H
        return c // H, lax.rem(c, H)

    def start_gather(c, buf):
        b, h = _bh(c)
        return pltpu.async_copy(
            table_hbm.at[idx_v.at[b, pl.ds(h * CG, CG)]],
            rows_v.at[buf], sems_g[buf])

    def start_scatter(c, buf):
        b, h = _bh(c)
        return pltpu.async_copy(
            rows_v.at[buf], out_hbm.at[b, pl.ds(s0 + h * CG, CG)],
            sems_o[buf])

    def wait_gather(buf):
        pltpu.make_async_copy(
            table_hbm.at[idx_v.at[0, pl.ds(0, CG)]],
            rows_v.at[buf], sems_g[buf]).wait()

    def wait_scatter(buf):
        pltpu.make_async_copy(
            rows_v.at[buf], out_hbm.at[0, pl.ds(0, CG)], sems_o[buf]).wait()

    def add_pe(c, buf):
        _, h = _bh(c)
        pe_base = h * CG

        @plsc.parallel_loop(0, CG, step=1, unroll=1)
        def _add_row(r):
            @plsc.parallel_loop(0, VECS, step=1, unroll=4)
            def _add_vec(v):
                sl = pl.ds(v * 16, 16)
                plsc.addupdate(rows_v.at[buf, r, sl], pe_v[pe_base + r, sl])

    for c in range(NAHEAD):
        start_gather(c, c % NBUF)
    cp_p.wait()

    def do_chunk(c, buf, tail_buf):
        @pl.when(c < NCHUNK)
        def _():
            wait_gather(buf)
            add_pe(c, buf)
            start_scatter(c, buf)

        nxt = c + NAHEAD

        @pl.when((nxt < NCHUNK) & (c >= NBUF - NAHEAD))
        def _():
            wait_scatter(tail_buf)

        @pl.when(nxt < NCHUNK)
        def _():
            start_gather(nxt, tail_buf)

    NSUPER = (NCHUNK + NBUF - 1) // NBUF

    def super_body(k, _):
        c0 = k * NBUF
        for j in range(NBUF):
            do_chunk(c0 + j, j, (j + NAHEAD) % NBUF)
        return ()

    lax.fori_loop(0, NSUPER, super_body, ())

    for c in range(NCHUNK - NBUF, NCHUNK):
        wait_scatter(c % NBUF)


@functools.cache
def _emb():
    mesh = plsc.VectorSubcoreMesh(core_axis_name="c", subcore_axis_name="s")
    return functools.partial(
        pl.kernel,
        mesh=mesh,
        out_type=jax.ShapeDtypeStruct((B, S, D), jnp.float32),
        scratch_types=[
            pltpu.VMEM((B, S_PER_W), jnp.int32),
            pltpu.VMEM((S_PER_W, D), jnp.float32),
            pltpu.VMEM((NBUF, CG, D), jnp.float32),
            pltpu.SemaphoreType.DMA,
            pltpu.SemaphoreType.DMA,
            [pltpu.SemaphoreType.DMA] * NBUF,
            [pltpu.SemaphoreType.DMA] * NBUF,
        ],
    )(_body)


@jax.jit
def kernel(x, table, pe):
    return _emb()(x, table, pe)


# speedup vs baseline: 1.0036x; 1.0036x over previous
import functools

import jax
import jax.numpy as jnp
from jax import lax
from jax.experimental import pallas as pl
from jax.experimental.pallas import tpu as pltpu
from jax.experimental.pallas import tpu_sc as plsc

B = 4
S = 2048
D = 768
NW = 32
S_PER_W = S // NW
CG = 32
H = S_PER_W // CG
NCHUNK = B * H
NBUF = 3
NAHEAD = 2
VECS = D // 16


def _body(x_hbm, table_hbm, pe_hbm, out_hbm, idx_v, pe_v, rows_v,
          sem_i, sem_p, sems_g, sems_o):
    cid = lax.axis_index("c")
    sid = lax.axis_index("s")
    wid = sid * 2 + cid
    s0 = wid * S_PER_W

    cps_i = [
        pltpu.async_copy(x_hbm.at[b, pl.ds(s0, S_PER_W)], idx_v.at[b], sem_i)
        for b in range(B)
    ]
    cp_p = pltpu.async_copy(pe_hbm.at[pl.ds(s0, S_PER_W)], pe_v, sem_p)
    for cp in cps_i:
        cp.wait()

    def _bh(c):
        if isinstance(c, int):
            return c // H, c % H
        return c // H, lax.rem(c, H)

    def start_gather(c, buf):
        b, h = _bh(c)
        return pltpu.async_copy(
            table_hbm.at[idx_v.at[b, pl.ds(h * CG, CG)]],
            rows_v.at[buf], sems_g[buf])

    def start_scatter(c, buf):
        b, h = _bh(c)
        return pltpu.async_copy(
            rows_v.at[buf], out_hbm.at[b, pl.ds(s0 + h * CG, CG)],
            sems_o[buf])

    def wait_gather(buf):
        pltpu.make_async_copy(
            table_hbm.at[idx_v.at[0, pl.ds(0, CG)]],
            rows_v.at[buf], sems_g[buf]).wait()

    def wait_scatter(buf):
        pltpu.make_async_copy(
            rows_v.at[buf], out_hbm.at[0, pl.ds(0, CG)], sems_o[buf]).wait()

    def add_pe(c, buf):
        _, h = _bh(c)
        pe_base = h * CG

        @plsc.parallel_loop(0, CG, step=1, unroll=1)
        def _add_row(r):
            @plsc.parallel_loop(0, VECS, step=1, unroll=4)
            def _add_vec(v):
                sl = pl.ds(v * 16, 16)
                plsc.addupdate(rows_v.at[buf, r, sl], pe_v[pe_base + r, sl])

    for c in range(NAHEAD):
        start_gather(c, c % NBUF)
    cp_p.wait()

    def do_chunk(c, buf, tail_buf):
        @pl.when(c < NCHUNK)
        def _():
            wait_gather(buf)
            add_pe(c, buf)
            start_scatter(c, buf)

        nxt = c + NAHEAD

        @pl.when((nxt < NCHUNK) & (c >= NBUF - NAHEAD))
        def _():
            wait_scatter(tail_buf)

        @pl.when(nxt < NCHUNK)
        def _():
            start_gather(nxt, tail_buf)

    NSUPER = (NCHUNK + NBUF - 1) // NBUF

    def super_body(k, _):
        c0 = k * NBUF
        for j in range(NBUF):
            do_chunk(c0 + j, j, (j + NAHEAD) % NBUF)
        return ()

    lax.fori_loop(0, NSUPER, super_body, ())

    for c in range(NCHUNK - NBUF, NCHUNK):
        wait_scatter(c % NBUF)


@functools.cache
def _emb():
    mesh = plsc.VectorSubcoreMesh(core_axis_name="c", subcore_axis_name="s")
    return functools.partial(
        pl.kernel,
        mesh=mesh,
        out_type=jax.ShapeDtypeStruct((B, S, D), jnp.float32),
        scratch_types=[
            pltpu.VMEM((B, S_PER_W), jnp.int32),
            pltpu.VMEM((S_PER_W, D), jnp.float32),
            pltpu.VMEM((NBUF, CG, D), jnp.float32),
            pltpu.SemaphoreType.DMA,
            pltpu.SemaphoreType.DMA,
            [pltpu.SemaphoreType.DMA] * NBUF,
            [pltpu.SemaphoreType.DMA] * NBUF,
        ],
    )(_body)


@jax.jit
def kernel(x, table, pe):
    return _emb()(x, table, pe)
